# baseline (speedup 1.0000x reference)
import functools

import numpy as np

import jax
import jax.numpy as jnp
from jax import lax
from jax.experimental import pallas as pl
from jax.experimental.pallas import tpu as pltpu
from jax.experimental.pallas import tpu_sc as plsc

N = 10000
E = 320000
D = 128
C = 128
ED = 16

NC = 2
NS = 16
NW = NC * NS
EPT = E // NW
B = 40
NCHUNK = EPT // B
RPT = 624
ZR = 16
LANES = 16
G_STARTS = (0, 16, 24)

_PERM = np.arange(C).reshape(C // 32, 2, 16).transpose(0, 2, 1).reshape(-1)


def _lin_body(x_ref, wl_ref, wr_ref, bl_ref, br_ref, xl_ref, xr_ref):
    x = x_ref[...]
    xl_ref[...] = (jnp.dot(x, wl_ref[...], preferred_element_type=jnp.float32)
                   + bl_ref[...]).astype(jnp.bfloat16)
    xr_ref[...] = (jnp.dot(x, wr_ref[...], preferred_element_type=jnp.float32)
                   + br_ref[...]).astype(jnp.bfloat16)


def _node_linears(x, W_l, W_r, b_l, b_r):
    blk = 1000
    return pl.pallas_call(
        _lin_body,
        grid=(N // blk,),
        in_specs=[
            pl.BlockSpec((blk, D), lambda i: (i, 0)),
            pl.BlockSpec((D, C), lambda i: (0, 0)),
            pl.BlockSpec((D, C), lambda i: (0, 0)),
            pl.BlockSpec((1, C), lambda i: (0, 0)),
            pl.BlockSpec((1, C), lambda i: (0, 0)),
        ],
        out_specs=[
            pl.BlockSpec((blk, C), lambda i: (i, 0)),
            pl.BlockSpec((blk, C), lambda i: (i, 0)),
        ],
        out_shape=[
            jax.ShapeDtypeStruct((N, C), jnp.bfloat16),
            jax.ShapeDtypeStruct((N, C), jnp.bfloat16),
        ],
    )(x, W_l, W_r, b_l, b_r)


def _edge_body(ea_ref, we_ref, e_ref):
    e_ref[...] = jnp.dot(ea_ref[...], we_ref[...],
                         preferred_element_type=jnp.float32).astype(jnp.bfloat16)


def _edge_linear(edge_attr, W_e):
    blk = 2000
    return pl.pallas_call(
        _edge_body,
        grid=(E // blk,),
        in_specs=[
            pl.BlockSpec((blk, ED), lambda i: (i, 0)),
            pl.BlockSpec((ED, C), lambda i: (0, 0)),
        ],
        out_specs=pl.BlockSpec((blk, C), lambda i: (i, 0)),
        out_shape=jax.ShapeDtypeStruct((E, C), jnp.bfloat16),
    )(edge_attr, W_e)


def _sc_edge_pass(xl, xr, e, sd, att):
    mesh = plsc.VectorSubcoreMesh(core_axis_name="c", subcore_axis_name="s")

    @functools.partial(
        pl.kernel,
        mesh=mesh,
        compiler_params=pltpu.CompilerParams(needs_layout_passes=False,
                                             use_tc_tiling_on_sc=False),
        out_type=[
            jax.ShapeDtypeStruct((NC * N, C), jnp.float32),
            jax.ShapeDtypeStruct((NW * N,), jnp.float32),
        ],
        scratch_types=[
            pltpu.VMEM((4, 2, B), jnp.int32),
            pltpu.VMEM((B, C // 2), jnp.int32),
            pltpu.VMEM((B, C // 2), jnp.int32),
            pltpu.VMEM((B, C // 2), jnp.int32),
            pltpu.VMEM((B, C // 2), jnp.int32),
            pltpu.VMEM((B, C // 2), jnp.int32),
            pltpu.VMEM((B, C // 2), jnp.int32),
            pltpu.VMEM((B, C), jnp.float32),
            pltpu.VMEM((B, C), jnp.float32),
            pltpu.VMEM((B,), jnp.float32),
            pltpu.VMEM((B * LANES,), jnp.float32),
            pltpu.VMEM((C,), jnp.bfloat16),
            pltpu.VMEM((N,), jnp.float32),
            pltpu.VMEM((ZR, C), jnp.float32),
            pltpu.VMEM_SHARED((N, C), jnp.float32),
            pltpu.SemaphoreType.DMA,
            pltpu.SemaphoreType.DMA,
            pltpu.SemaphoreType.DMA,
            pltpu.SemaphoreType.DMA,
            pltpu.SemaphoreType.DMA,
        ],
    )
    def body(xl_hbm, xr_hbm, e_hbm, sd_hbm, att_hbm,
             outp_hbm, denp_hbm,
             idx2, rl0, rl1, rr0, rr1, re0, re1, st0, st1, ea_v, accb_v,
             att_v, den_v, z_v, out_sh, gsem0, gsem1, ssem, isem0, isem1):
        cid = lax.axis_index("c")
        sid = lax.axis_index("s")
        wid = cid * NS + sid
        rl = (rl0, rl1)
        rr = (rr0, rr1)
        re = (re0, re1)
        st = (st0, st1)
        gsem = (gsem0, gsem1)

        pltpu.sync_copy(att_hbm, att_v)

        def zero_z(r, _):
            for j in range(C // LANES):
                z_v[r, pl.ds(j * LANES, LANES)] = jnp.zeros((LANES,), jnp.float32)
            return _
        lax.fori_loop(0, ZR, zero_z, None)

        def zero_den(g, _):
            den_v[pl.ds(g * LANES, LANES)] = jnp.zeros((LANES,), jnp.float32)
            return _
        lax.fori_loop(0, N // LANES, zero_den, None)

        def zero_sh(k, _):
            pltpu.sync_copy(z_v, out_sh.at[pl.ds(sid * RPT + k * ZR, ZR)])
            return _
        lax.fori_loop(0, RPT // ZR, zero_sh, None)

        @pl.when(sid == NS - 1)
        def _():
            pltpu.sync_copy(z_v.at[pl.ds(0, 16)],
                            out_sh.at[pl.ds(NS * RPT, 16)])

        plsc.subcore_barrier()

        isem = (isem0, isem1)

        def fetch_idx(c, s, par):
            pltpu.async_copy(sd_hbm.at[wid, c], idx2.at[s], isem[par])

        def wait_idx(s, par):
            pltpu.make_async_copy(sd_hbm.at[wid, 0], idx2.at[s],
                                  isem[par]).wait()

        def issue_rows(c, s, p):
            pltpu.async_copy(xl_hbm.at[idx2.at[s, 0]], rl[p], gsem[p])
            pltpu.async_copy(xr_hbm.at[idx2.at[s, 1]], rr[p], gsem[p])
            pltpu.async_copy(e_hbm.at[pl.ds(wid * EPT + c * B, B)],
                             re[p], gsem[p])

        def wait_rows(p):
            pltpu.make_async_copy(e_hbm.at[pl.ds(0, B)], rl[p], gsem[p]).wait()
            pltpu.make_async_copy(e_hbm.at[pl.ds(0, B)], rr[p], gsem[p]).wait()
            pltpu.make_async_copy(e_hbm.at[pl.ds(0, B)], re[p], gsem[p]).wait()

        def issue_scat(s, p):
            pltpu.async_copy(st[p], out_sh.at[idx2.at[s, 1]], ssem,
                             add=True)

        def wait_scat(s, p):
            pltpu.make_async_copy(st[p], out_sh.at[idx2.at[s, 1]],
                                  ssem).wait()

        lane = lax.iota(jnp.int32, LANES)
        tail_mask = lane >= (G_STARTS[-2] + LANES - G_STARTS[-1])

        def compute(s, p):
            rl_v, rr_v, re_v, st_v = rl[p], rr[p], re[p], st[p]

            def edge_alpha(eg, _):
                acc = jnp.zeros((LANES,), jnp.float32)
                for j in range(C // 32):
                    sl = pl.ds(j * LANES, LANES)
                    lv = plsc.bitcast(rl_v[eg, sl], jnp.bfloat16)
                    rv = plsc.bitcast(rr_v[eg, sl], jnp.bfloat16)
                    ev = plsc.bitcast(re_v[eg, sl], jnp.bfloat16)
                    m = lv + rv + ev
                    m = jnp.maximum(m, jnp.bfloat16(0.2) * m)
                    t = m * att_v[pl.ds(j * 32, 32)]
                    u, v = plsc.unpack(t, format=plsc.PackFormat.INTERLEAVED)
                    acc = acc + u + v
                accb_v[pl.ds(eg * LANES, LANES)] = acc
                return _
            lax.fori_loop(0, B, edge_alpha, None)

            for g0 in G_STARTS:
                sl = pl.ds(g0, LANES)
                flat = (lane + g0) * LANES
                asum = jnp.zeros((LANES,), jnp.float32)
                for cc in range(LANES):
                    asum = asum + plsc.load_gather(accb_v, [flat + cc])
                eav = jnp.exp(asum)
                ea_v[sl] = eav
                mask = tail_mask if g0 == G_STARTS[-1] else None
                plsc.addupdate_scatter(den_v, [idx2[s, 1, sl]], eav,
                                       mask=mask)

            def edge_scale(eg, _):
                sc = plsc.load_gather(ea_v, [jnp.full((LANES,), eg, jnp.int32)])
                for j in range(C // 32):
                    x32 = plsc.bitcast(rl_v[eg, pl.ds(j * LANES, LANES)],
                                       jnp.bfloat16)
                    u, v = plsc.unpack(x32, format=plsc.PackFormat.INTERLEAVED)
                    st_v[eg, pl.ds(j * 32, LANES)] = u * sc
                    st_v[eg, pl.ds(j * 32 + LANES, LANES)] = v * sc
                return _
            lax.fori_loop(0, B, edge_scale, None)

        fetch_idx(0, 0, 0)
        wait_idx(0, 0)
        issue_rows(0, 0, 0)
        fetch_idx(1, 1, 1)

        def pipe_body(k):
            for b in range(4):
                c = k + b
                s = b
                sn = (b + 1) % 4
                sf = (b + 2) % 4
                p = b % 2
                q = 1 - p
                wait_rows(p)

                @pl.when(c >= 1)
                def _():
                    wait_scat((b + 3) % 4, q)
                wait_idx(sn, (b + 1) % 2)
                issue_rows(c + 1, sn, q)
                fetch_idx(c + 2, sf, b % 2)
                compute(s, p)
                issue_scat(s, p)

        pl.loop(0, NCHUNK - 2, step=4)(pipe_body)

        wait_rows(0)
        wait_scat(3, 1)
        wait_idx(1, 1)
        issue_rows(NCHUNK - 1, 1, 1)
        compute(0, 0)
        issue_scat(0, 0)

        wait_rows(1)
        wait_scat(0, 0)
        compute(1, 1)
        issue_scat(1, 1)
        wait_scat(1, 1)

        plsc.subcore_barrier()

        pltpu.sync_copy(out_sh.at[pl.ds(sid * RPT, RPT)],
                        outp_hbm.at[pl.ds(cid * N + sid * RPT, RPT)])

        @pl.when(sid == NS - 1)
        def _():
            pltpu.sync_copy(out_sh.at[pl.ds(NS * RPT, 16)],
                            outp_hbm.at[pl.ds(cid * N + NS * RPT, 16)])

        pltpu.sync_copy(den_v, denp_hbm.at[pl.ds(wid * N, N)])

    return body(xl, xr, e, sd, att)


def _comb_body(p_ref, den_ref, bias_ref, y_ref, ps_ref, pq_ref):
    p = p_ref[0] + p_ref[1]
    den = jnp.sum(den_ref[...], axis=1, keepdims=True)
    y = p / (den + 1e-16) + bias_ref[...]
    y_ref[...] = y
    ps_ref[...] = jnp.broadcast_to(jnp.sum(y, axis=0, keepdims=True), (8, C))
    pq_ref[...] = jnp.broadcast_to(jnp.sum(y * y, axis=0, keepdims=True), (8, C))


def _combine(p, denp, bias):
    blk = 1000
    nb = N // blk
    return pl.pallas_call(
        _comb_body,
        grid=(nb,),
        in_specs=[
            pl.BlockSpec((NC, blk, C), lambda i: (0, i, 0)),
            pl.BlockSpec((blk, NW), lambda i: (i, 0)),
            pl.BlockSpec((1, C), lambda i: (0, 0)),
        ],
        out_specs=[
            pl.BlockSpec((blk, C), lambda i: (i, 0)),
            pl.BlockSpec((8, C), lambda i: (i, 0)),
            pl.BlockSpec((8, C), lambda i: (i, 0)),
        ],
        out_shape=[
            jax.ShapeDtypeStruct((N, C), jnp.float32),
            jax.ShapeDtypeStruct((nb * 8, C), jnp.float32),
            jax.ShapeDtypeStruct((nb * 8, C), jnp.float32),
        ],
    )(p, denp, bias)


def _bn_body(y_ref, ps_ref, pq_ref, gamma_ref, beta_ref, out_ref):
    s = jnp.sum(ps_ref[...], axis=0, keepdims=True)
    sq = jnp.sum(pq_ref[...], axis=0, keepdims=True)
    mean = s / (8.0 * N)
    var = sq / (8.0 * N) - mean * mean
    inv = lax.rsqrt(var + 1e-5)
    o = (y_ref[...] - mean) * (inv * gamma_ref[...]) + beta_ref[...]
    out_ref[...] = jnp.where(o > 0, o, 0.01 * o)


def _batchnorm(y, ps, pq, gamma, beta):
    blk = 1000
    nb = N // blk
    return pl.pallas_call(
        _bn_body,
        grid=(nb,),
        in_specs=[
            pl.BlockSpec((blk, C), lambda i: (i, 0)),
            pl.BlockSpec((nb * 8, C), lambda i: (0, 0)),
            pl.BlockSpec((nb * 8, C), lambda i: (0, 0)),
            pl.BlockSpec((1, C), lambda i: (0, 0)),
            pl.BlockSpec((1, C), lambda i: (0, 0)),
        ],
        out_specs=pl.BlockSpec((blk, C), lambda i: (i, 0)),
        out_shape=jax.ShapeDtypeStruct((N, C), jnp.float32),
    )(y, ps, pq, gamma, beta)


def kernel(x, edge_index, edge_attr, W_l, b_l, W_r, b_r, W_e, att, bias, gamma, beta):
    xl, xr = _node_linears(x, W_l[:, _PERM], W_r[:, _PERM],
                           b_l[_PERM].reshape(1, C), b_r[_PERM].reshape(1, C))
    e = _edge_linear(edge_attr, W_e[:, _PERM])
    sd = jnp.stack([edge_index[0].reshape(NW, NCHUNK, B),
                    edge_index[1].reshape(NW, NCHUNK, B)], axis=2)
    xl_i = lax.bitcast_convert_type(xl.reshape(N, C // 2, 2), jnp.int32)
    xr_i = lax.bitcast_convert_type(xr.reshape(N, C // 2, 2), jnp.int32)
    e_i = lax.bitcast_convert_type(e.reshape(E, C // 2, 2), jnp.int32)
    outp, denp = _sc_edge_pass(xl_i, xr_i, e_i, sd,
                               att[_PERM].astype(jnp.bfloat16))
    y, ps, pq = _combine(outp.reshape(NC, N, C), denp.reshape(NW, N).T,
                         bias.reshape(1, C))
    return _batchnorm(y, ps, pq, gamma.reshape(1, C), beta.reshape(1, C))


# speedup vs baseline: 1.7914x; 1.7914x over previous
import functools

import numpy as np

import jax
import jax.numpy as jnp
from jax import lax
from jax.experimental import pallas as pl
from jax.experimental.pallas import tpu as pltpu
from jax.experimental.pallas import tpu_sc as plsc

N = 10000
E = 320000
D = 128
C = 128
ED = 16

NC = 2
NS = 16
NW = NC * NS
EPT = E // NW
B = 40
NCHUNK = EPT // B
RPT = 624
ZR = 16
LANES = 16
G_STARTS = (0, 16, 24)

_ATT_PERM = np.arange(C).reshape(2, C // 32, 16).transpose(1, 2, 0).reshape(-1)


def _pack_i32(y):
    h = lax.bitcast_convert_type(y.astype(jnp.bfloat16), jnp.uint16)
    hw = h.astype(jnp.uint32)
    word = hw[:, : C // 2] | (hw[:, C // 2:] << 16)
    return lax.bitcast_convert_type(word, jnp.int32)


def _lin_body(x_ref, wl_ref, wr_ref, bl_ref, br_ref, xl_ref, xr_ref):
    x = x_ref[...]
    xl_ref[...] = _pack_i32(
        jnp.dot(x, wl_ref[...], preferred_element_type=jnp.float32)
        + bl_ref[...])
    xr_ref[...] = _pack_i32(
        jnp.dot(x, wr_ref[...], preferred_element_type=jnp.float32)
        + br_ref[...])


def _node_linears(x, W_l, W_r, b_l, b_r):
    blk = 1000
    return pl.pallas_call(
        _lin_body,
        grid=(N // blk,),
        in_specs=[
            pl.BlockSpec((blk, D), lambda i: (i, 0)),
            pl.BlockSpec((D, C), lambda i: (0, 0)),
            pl.BlockSpec((D, C), lambda i: (0, 0)),
            pl.BlockSpec((1, C), lambda i: (0, 0)),
            pl.BlockSpec((1, C), lambda i: (0, 0)),
        ],
        out_specs=[
            pl.BlockSpec((blk, C // 2), lambda i: (i, 0)),
            pl.BlockSpec((blk, C // 2), lambda i: (i, 0)),
        ],
        out_shape=[
            jax.ShapeDtypeStruct((N, C // 2), jnp.int32),
            jax.ShapeDtypeStruct((N, C // 2), jnp.int32),
        ],
    )(x, W_l, W_r, b_l, b_r)


def _edge_body(ea_ref, we_ref, e_ref):
    e_ref[...] = _pack_i32(jnp.dot(ea_ref[...], we_ref[...],
                                   preferred_element_type=jnp.float32))


def _edge_linear(edge_attr, W_e):
    blk = 2000
    return pl.pallas_call(
        _edge_body,
        grid=(E // blk,),
        in_specs=[
            pl.BlockSpec((blk, ED), lambda i: (i, 0)),
            pl.BlockSpec((ED, C), lambda i: (0, 0)),
        ],
        out_specs=pl.BlockSpec((blk, C // 2), lambda i: (i, 0)),
        out_shape=jax.ShapeDtypeStruct((E, C // 2), jnp.int32),
    )(edge_attr, W_e)


def _sc_edge_pass(xl, xr, e, sd, att):
    mesh = plsc.VectorSubcoreMesh(core_axis_name="c", subcore_axis_name="s")

    @functools.partial(
        pl.kernel,
        mesh=mesh,
        compiler_params=pltpu.CompilerParams(needs_layout_passes=False,
                                             use_tc_tiling_on_sc=False),
        out_type=[
            jax.ShapeDtypeStruct((NC * N, C), jnp.float32),
            jax.ShapeDtypeStruct((NW * N,), jnp.float32),
        ],
        scratch_types=[
            pltpu.VMEM((4, 2, B), jnp.int32),
            pltpu.VMEM((B, C // 2), jnp.int32),
            pltpu.VMEM((B, C // 2), jnp.int32),
            pltpu.VMEM((B, C // 2), jnp.int32),
            pltpu.VMEM((B, C // 2), jnp.int32),
            pltpu.VMEM((B, C // 2), jnp.int32),
            pltpu.VMEM((B, C // 2), jnp.int32),
            pltpu.VMEM((B, C), jnp.float32),
            pltpu.VMEM((B, C), jnp.float32),
            pltpu.VMEM((B,), jnp.float32),
            pltpu.VMEM((B * LANES,), jnp.float32),
            pltpu.VMEM((C,), jnp.bfloat16),
            pltpu.VMEM((N,), jnp.float32),
            pltpu.VMEM((ZR, C), jnp.float32),
            pltpu.VMEM_SHARED((N, C), jnp.float32),
            pltpu.SemaphoreType.DMA,
            pltpu.SemaphoreType.DMA,
            pltpu.SemaphoreType.DMA,
            pltpu.SemaphoreType.DMA,
            pltpu.SemaphoreType.DMA,
        ],
    )
    def body(xl_hbm, xr_hbm, e_hbm, sd_hbm, att_hbm,
             outp_hbm, denp_hbm,
             idx2, rl0, rl1, rr0, rr1, re0, re1, st0, st1, ea_v, accb_v,
             att_v, den_v, z_v, out_sh, gsem0, gsem1, ssem, isem0, isem1):
        cid = lax.axis_index("c")
        sid = lax.axis_index("s")
        wid = cid * NS + sid
        rl = (rl0, rl1)
        rr = (rr0, rr1)
        re = (re0, re1)
        st = (st0, st1)
        gsem = (gsem0, gsem1)

        pltpu.sync_copy(att_hbm, att_v)

        def zero_z(r, _):
            for j in range(C // LANES):
                z_v[r, pl.ds(j * LANES, LANES)] = jnp.zeros((LANES,), jnp.float32)
            return _
        lax.fori_loop(0, ZR, zero_z, None)

        def zero_den(g, _):
            den_v[pl.ds(g * LANES, LANES)] = jnp.zeros((LANES,), jnp.float32)
            return _
        lax.fori_loop(0, N // LANES, zero_den, None)

        def zero_sh(k, _):
            pltpu.sync_copy(z_v, out_sh.at[pl.ds(sid * RPT + k * ZR, ZR)])
            return _
        lax.fori_loop(0, RPT // ZR, zero_sh, None)

        @pl.when(sid == NS - 1)
        def _():
            pltpu.sync_copy(z_v.at[pl.ds(0, 16)],
                            out_sh.at[pl.ds(NS * RPT, 16)])

        plsc.subcore_barrier()

        isem = (isem0, isem1)

        def fetch_idx(c, s, par):
            pltpu.async_copy(sd_hbm.at[wid, c], idx2.at[s], isem[par])

        def wait_idx(s, par):
            pltpu.make_async_copy(sd_hbm.at[wid, 0], idx2.at[s],
                                  isem[par]).wait()

        def issue_rows(c, s, p):
            pltpu.async_copy(xl_hbm.at[idx2.at[s, 0]], rl[p], gsem[p])
            pltpu.async_copy(xr_hbm.at[idx2.at[s, 1]], rr[p], gsem[p])
            pltpu.async_copy(e_hbm.at[pl.ds(wid * EPT + c * B, B)],
                             re[p], gsem[p])

        def wait_rows(p):
            pltpu.make_async_copy(e_hbm.at[pl.ds(0, B)], rl[p], gsem[p]).wait()
            pltpu.make_async_copy(e_hbm.at[pl.ds(0, B)], rr[p], gsem[p]).wait()
            pltpu.make_async_copy(e_hbm.at[pl.ds(0, B)], re[p], gsem[p]).wait()

        def issue_scat(s, p):
            pltpu.async_copy(st[p], out_sh.at[idx2.at[s, 1]], ssem,
                             add=True)

        def wait_scat(s, p):
            pltpu.make_async_copy(st[p], out_sh.at[idx2.at[s, 1]],
                                  ssem).wait()

        lane = lax.iota(jnp.int32, LANES)
        tail_mask = lane >= (G_STARTS[-2] + LANES - G_STARTS[-1])

        def compute(s, p):
            rl_v, rr_v, re_v, st_v = rl[p], rr[p], re[p], st[p]

            def edge_alpha(eg, _):
                acc = jnp.zeros((LANES,), jnp.float32)
                for j in range(C // 32):
                    sl = pl.ds(j * LANES, LANES)
                    lv = plsc.bitcast(rl_v[eg, sl], jnp.bfloat16)
                    rv = plsc.bitcast(rr_v[eg, sl], jnp.bfloat16)
                    ev = plsc.bitcast(re_v[eg, sl], jnp.bfloat16)
                    m = lv + rv + ev
                    m = jnp.maximum(m, jnp.bfloat16(0.2) * m)
                    t = m * att_v[pl.ds(j * 32, 32)]
                    u, v = plsc.unpack(t, format=plsc.PackFormat.INTERLEAVED)
                    acc = acc + u + v
                accb_v[pl.ds(eg * LANES, LANES)] = acc
                return _
            lax.fori_loop(0, B, edge_alpha, None)

            for g0 in G_STARTS:
                sl = pl.ds(g0, LANES)
                flat = (lane + g0) * LANES
                asum = jnp.zeros((LANES,), jnp.float32)
                for cc in range(LANES):
                    asum = asum + plsc.load_gather(accb_v, [flat + cc])
                eav = jnp.exp(asum)
                ea_v[sl] = eav
                mask = tail_mask if g0 == G_STARTS[-1] else None
                plsc.addupdate_scatter(den_v, [idx2[s, 1, sl]], eav,
                                       mask=mask)

            def edge_scale(eg, _):
                sc = plsc.load_gather(ea_v, [jnp.full((LANES,), eg, jnp.int32)])
                for j in range(C // 32):
                    x32 = plsc.bitcast(rl_v[eg, pl.ds(j * LANES, LANES)],
                                       jnp.bfloat16)
                    u, v = plsc.unpack(x32, format=plsc.PackFormat.INTERLEAVED)
                    st_v[eg, pl.ds(j * LANES, LANES)] = u * sc
                    st_v[eg, pl.ds(C // 2 + j * LANES, LANES)] = v * sc
                return _
            lax.fori_loop(0, B, edge_scale, None)

        fetch_idx(0, 0, 0)
        wait_idx(0, 0)
        issue_rows(0, 0, 0)
        fetch_idx(1, 1, 1)

        def pipe_body(k):
            for b in range(4):
                c = k + b
                s = b
                sn = (b + 1) % 4
                sf = (b + 2) % 4
                p = b % 2
                q = 1 - p
                wait_rows(p)

                @pl.when(c >= 1)
                def _():
                    wait_scat((b + 3) % 4, q)
                wait_idx(sn, (b + 1) % 2)
                issue_rows(c + 1, sn, q)
                fetch_idx(c + 2, sf, b % 2)
                compute(s, p)
                issue_scat(s, p)

        pl.loop(0, NCHUNK - 2, step=4)(pipe_body)

        wait_rows(0)
        wait_scat(3, 1)
        wait_idx(1, 1)
        issue_rows(NCHUNK - 1, 1, 1)
        compute(0, 0)
        issue_scat(0, 0)

        wait_rows(1)
        wait_scat(0, 0)
        compute(1, 1)
        issue_scat(1, 1)
        wait_scat(1, 1)

        plsc.subcore_barrier()

        pltpu.sync_copy(out_sh.at[pl.ds(sid * RPT, RPT)],
                        outp_hbm.at[pl.ds(cid * N + sid * RPT, RPT)])

        @pl.when(sid == NS - 1)
        def _():
            pltpu.sync_copy(out_sh.at[pl.ds(NS * RPT, 16)],
                            outp_hbm.at[pl.ds(cid * N + NS * RPT, 16)])

        pltpu.sync_copy(den_v, denp_hbm.at[pl.ds(wid * N, N)])

    return body(xl, xr, e, sd, att)


def _comb_body(p_ref, den_ref, bias_ref, y_ref, ps_ref, pq_ref):
    p = p_ref[0] + p_ref[1]
    den = jnp.sum(den_ref[...], axis=1, keepdims=True)
    y = p / (den + 1e-16) + bias_ref[...]
    y_ref[...] = y
    ps_ref[...] = jnp.broadcast_to(jnp.sum(y, axis=0, keepdims=True), (8, C))
    pq_ref[...] = jnp.broadcast_to(jnp.sum(y * y, axis=0, keepdims=True), (8, C))


def _combine(p, denp, bias):
    blk = 1000
    nb = N // blk
    return pl.pallas_call(
        _comb_body,
        grid=(nb,),
        in_specs=[
            pl.BlockSpec((NC, blk, C), lambda i: (0, i, 0)),
            pl.BlockSpec((blk, NW), lambda i: (i, 0)),
            pl.BlockSpec((1, C), lambda i: (0, 0)),
        ],
        out_specs=[
            pl.BlockSpec((blk, C), lambda i: (i, 0)),
            pl.BlockSpec((8, C), lambda i: (i, 0)),
            pl.BlockSpec((8, C), lambda i: (i, 0)),
        ],
        out_shape=[
            jax.ShapeDtypeStruct((N, C), jnp.float32),
            jax.ShapeDtypeStruct((nb * 8, C), jnp.float32),
            jax.ShapeDtypeStruct((nb * 8, C), jnp.float32),
        ],
    )(p, denp, bias)


def _bn_body(y_ref, ps_ref, pq_ref, gamma_ref, beta_ref, out_ref):
    s = jnp.sum(ps_ref[...], axis=0, keepdims=True)
    sq = jnp.sum(pq_ref[...], axis=0, keepdims=True)
    mean = s / (8.0 * N)
    var = sq / (8.0 * N) - mean * mean
    inv = lax.rsqrt(var + 1e-5)
    o = (y_ref[...] - mean) * (inv * gamma_ref[...]) + beta_ref[...]
    out_ref[...] = jnp.where(o > 0, o, 0.01 * o)


def _batchnorm(y, ps, pq, gamma, beta):
    blk = 1000
    nb = N // blk
    return pl.pallas_call(
        _bn_body,
        grid=(nb,),
        in_specs=[
            pl.BlockSpec((blk, C), lambda i: (i, 0)),
            pl.BlockSpec((nb * 8, C), lambda i: (0, 0)),
            pl.BlockSpec((nb * 8, C), lambda i: (0, 0)),
            pl.BlockSpec((1, C), lambda i: (0, 0)),
            pl.BlockSpec((1, C), lambda i: (0, 0)),
        ],
        out_specs=pl.BlockSpec((blk, C), lambda i: (i, 0)),
        out_shape=jax.ShapeDtypeStruct((N, C), jnp.float32),
    )(y, ps, pq, gamma, beta)


def kernel(x, edge_index, edge_attr, W_l, b_l, W_r, b_r, W_e, att, bias, gamma, beta):
    xl, xr = _node_linears(x, W_l, W_r,
                           b_l.reshape(1, C), b_r.reshape(1, C))
    e = _edge_linear(edge_attr, W_e)
    sd = jnp.stack([edge_index[0].reshape(NW, NCHUNK, B),
                    edge_index[1].reshape(NW, NCHUNK, B)], axis=2)
    outp, denp = _sc_edge_pass(xl, xr, e, sd,
                               att[_ATT_PERM].astype(jnp.bfloat16))
    y, ps, pq = _combine(outp.reshape(NC, N, C), denp.reshape(NW, N).T,
                         bias.reshape(1, C))
    return _batchnorm(y, ps, pq, gamma.reshape(1, C), beta.reshape(1, C))


# speedup vs baseline: 2.0130x; 1.1237x over previous
import functools

import numpy as np

import jax
import jax.numpy as jnp
from jax import lax
from jax.experimental import pallas as pl
from jax.experimental.pallas import tpu as pltpu
from jax.experimental.pallas import tpu_sc as plsc

N = 10000
E = 320000
D = 128
C = 128
ED = 16

NC = 2
NS = 16
NW = NC * NS
EPT = E // NW
B = 40
NCHUNK = EPT // B
RPT = 624
ZR = 16
LANES = 16
G_STARTS = (0, 16, 24)

_ATT_PERM = np.arange(C).reshape(2, C // 32, 16).transpose(1, 2, 0).reshape(-1)


def _pack_i32(y):
    h = lax.bitcast_convert_type(y.astype(jnp.bfloat16), jnp.uint16)
    hw = h.astype(jnp.uint32)
    word = hw[:, : C // 2] | (hw[:, C // 2:] << 16)
    return lax.bitcast_convert_type(word, jnp.int32)


def _lin_body(x_ref, wl_ref, wr_ref, bl_ref, br_ref, xl_ref, xr_ref):
    x = x_ref[...]
    xl_ref[...] = (jnp.dot(x, wl_ref[...], preferred_element_type=jnp.float32)
                   + bl_ref[...])
    xr_ref[...] = _pack_i32(
        jnp.dot(x, wr_ref[...], preferred_element_type=jnp.float32)
        + br_ref[...])


def _node_linears(x, W_l, W_r, b_l, b_r):
    blk = 1000
    return pl.pallas_call(
        _lin_body,
        grid=(N // blk,),
        in_specs=[
            pl.BlockSpec((blk, D), lambda i: (i, 0)),
            pl.BlockSpec((D, C), lambda i: (0, 0)),
            pl.BlockSpec((D, C), lambda i: (0, 0)),
            pl.BlockSpec((1, C), lambda i: (0, 0)),
            pl.BlockSpec((1, C), lambda i: (0, 0)),
        ],
        out_specs=[
            pl.BlockSpec((blk, C), lambda i: (i, 0)),
            pl.BlockSpec((blk, C // 2), lambda i: (i, 0)),
        ],
        out_shape=[
            jax.ShapeDtypeStruct((N, C), jnp.float32),
            jax.ShapeDtypeStruct((N, C // 2), jnp.int32),
        ],
    )(x, W_l, W_r, b_l, b_r)


def _edge_body(ea_ref, we_ref, e_ref):
    e_ref[...] = _pack_i32(jnp.dot(ea_ref[...], we_ref[...],
                                   preferred_element_type=jnp.float32))


def _edge_linear(edge_attr, W_e):
    blk = 2000
    return pl.pallas_call(
        _edge_body,
        grid=(E // blk,),
        in_specs=[
            pl.BlockSpec((blk, ED), lambda i: (i, 0)),
            pl.BlockSpec((ED, C), lambda i: (0, 0)),
        ],
        out_specs=pl.BlockSpec((blk, C // 2), lambda i: (i, 0)),
        out_shape=jax.ShapeDtypeStruct((E, C // 2), jnp.int32),
    )(edge_attr, W_e)


def _sc_edge_pass(xl, xr, e, sd, att):
    mesh = plsc.VectorSubcoreMesh(core_axis_name="c", subcore_axis_name="s")

    @functools.partial(
        pl.kernel,
        mesh=mesh,
        compiler_params=pltpu.CompilerParams(needs_layout_passes=False,
                                             use_tc_tiling_on_sc=False),
        out_type=[
            jax.ShapeDtypeStruct((NC * N, C), jnp.float32),
            jax.ShapeDtypeStruct((NW * N,), jnp.float32),
        ],
        scratch_types=[
            pltpu.VMEM((4, 2, B), jnp.int32),
            pltpu.VMEM((B, C), jnp.float32),
            pltpu.VMEM((B, C), jnp.float32),
            pltpu.VMEM((B, C // 2), jnp.int32),
            pltpu.VMEM((B, C // 2), jnp.int32),
            pltpu.VMEM((B, C // 2), jnp.int32),
            pltpu.VMEM((B, C // 2), jnp.int32),
            pltpu.VMEM((B,), jnp.float32),
            pltpu.VMEM((B * LANES,), jnp.float32),
            pltpu.VMEM((C,), jnp.float32),
            pltpu.VMEM((N,), jnp.float32),
            pltpu.VMEM((ZR, C), jnp.float32),
            pltpu.VMEM_SHARED((N, C), jnp.float32),
            pltpu.SemaphoreType.DMA,
            pltpu.SemaphoreType.DMA,
            pltpu.SemaphoreType.DMA,
            pltpu.SemaphoreType.DMA,
            pltpu.SemaphoreType.DMA,
        ],
    )
    def body(xl_hbm, xr_hbm, e_hbm, sd_hbm, att_hbm,
             outp_hbm, denp_hbm,
             idx2, rl0, rl1, rr0, rr1, re0, re1, ea_v, accb_v,
             att_v, den_v, z_v, out_sh, gsem0, gsem1, ssem, isem0, isem1):
        cid = lax.axis_index("c")
        sid = lax.axis_index("s")
        wid = cid * NS + sid
        rl = (rl0, rl1)
        rr = (rr0, rr1)
        re = (re0, re1)
        gsem = (gsem0, gsem1)

        pltpu.sync_copy(att_hbm, att_v)

        def zero_z(r, _):
            for j in range(C // LANES):
                z_v[r, pl.ds(j * LANES, LANES)] = jnp.zeros((LANES,), jnp.float32)
            return _
        lax.fori_loop(0, ZR, zero_z, None)

        def zero_den(g, _):
            den_v[pl.ds(g * LANES, LANES)] = jnp.zeros((LANES,), jnp.float32)
            return _
        lax.fori_loop(0, N // LANES, zero_den, None)

        def zero_sh(k, _):
            pltpu.sync_copy(z_v, out_sh.at[pl.ds(sid * RPT + k * ZR, ZR)])
            return _
        lax.fori_loop(0, RPT // ZR, zero_sh, None)

        @pl.when(sid == NS - 1)
        def _():
            pltpu.sync_copy(z_v.at[pl.ds(0, 16)],
                            out_sh.at[pl.ds(NS * RPT, 16)])

        plsc.subcore_barrier()

        isem = (isem0, isem1)

        def fetch_idx(c, s, par):
            pltpu.async_copy(sd_hbm.at[wid, c], idx2.at[s], isem[par])

        def wait_idx(s, par):
            pltpu.make_async_copy(sd_hbm.at[wid, 0], idx2.at[s],
                                  isem[par]).wait()

        def issue_rows(c, s, p):
            pltpu.async_copy(xl_hbm.at[idx2.at[s, 0]], rl[p], gsem[p])
            pltpu.async_copy(xr_hbm.at[idx2.at[s, 1]], rr[p], gsem[p])
            pltpu.async_copy(e_hbm.at[pl.ds(wid * EPT + c * B, B)],
                             re[p], gsem[p])

        def wait_rows(p):
            pltpu.make_async_copy(e_hbm.at[pl.ds(0, B)], rl[p], gsem[p]).wait()
            pltpu.make_async_copy(e_hbm.at[pl.ds(0, B)], rr[p], gsem[p]).wait()
            pltpu.make_async_copy(e_hbm.at[pl.ds(0, B)], re[p], gsem[p]).wait()

        def issue_scat(s, p):
            pltpu.async_copy(rl[p], out_sh.at[idx2.at[s, 1]], ssem,
                             add=True)

        def wait_scat(s, p):
            pltpu.make_async_copy(rl[p], out_sh.at[idx2.at[s, 1]],
                                  ssem).wait()

        lane = lax.iota(jnp.int32, LANES)
        tail_mask = lane >= (G_STARTS[-2] + LANES - G_STARTS[-1])

        def compute(s, p):
            rl_v, rr_v, re_v = rl[p], rr[p], re[p]

            def edge_alpha(eg, _):
                acc = jnp.zeros((LANES,), jnp.float32)
                for j in range(C // 32):
                    slw = pl.ds(j * LANES, LANES)
                    ur, vr = plsc.unpack(
                        plsc.bitcast(rr_v[eg, slw], jnp.bfloat16),
                        format=plsc.PackFormat.INTERLEAVED)
                    ue, ve = plsc.unpack(
                        plsc.bitcast(re_v[eg, slw], jnp.bfloat16),
                        format=plsc.PackFormat.INTERLEAVED)
                    for (half, rv, ev) in ((0, ur, ue), (C // 2, vr, ve)):
                        sl = pl.ds(half + j * LANES, LANES)
                        m = rl_v[eg, sl] + rv + ev
                        m = jnp.maximum(m, 0.2 * m)
                        acc = acc + m * att_v[sl]
                accb_v[pl.ds(eg * LANES, LANES)] = acc
                return _
            lax.fori_loop(0, B, edge_alpha, None)

            for g0 in G_STARTS:
                sl = pl.ds(g0, LANES)
                flat = (lane + g0) * LANES
                asum = jnp.zeros((LANES,), jnp.float32)
                for cc in range(LANES):
                    asum = asum + plsc.load_gather(accb_v, [flat + cc])
                eav = jnp.exp(asum)
                ea_v[sl] = eav
                mask = tail_mask if g0 == G_STARTS[-1] else None
                plsc.addupdate_scatter(den_v, [idx2[s, 1, sl]], eav,
                                       mask=mask)

            def edge_scale(eg, _):
                sc = plsc.load_gather(ea_v, [jnp.full((LANES,), eg, jnp.int32)])
                for j in range(C // LANES):
                    sl = pl.ds(j * LANES, LANES)
                    rl_v[eg, sl] = rl_v[eg, sl] * sc
                return _
            lax.fori_loop(0, B, edge_scale, None)

        fetch_idx(0, 0, 0)
        wait_idx(0, 0)
        issue_rows(0, 0, 0)
        fetch_idx(1, 1, 1)

        def pipe_body(k):
            for b in range(4):
                c = k + b
                s = b
                sn = (b + 1) % 4
                sf = (b + 2) % 4
                p = b % 2
                q = 1 - p
                wait_rows(p)

                @pl.when(c >= 1)
                def _():
                    wait_scat((b + 3) % 4, q)
                wait_idx(sn, (b + 1) % 2)
                issue_rows(c + 1, sn, q)
                fetch_idx(c + 2, sf, b % 2)
                compute(s, p)
                issue_scat(s, p)

        pl.loop(0, NCHUNK - 2, step=4)(pipe_body)

        wait_rows(0)
        wait_scat(3, 1)
        wait_idx(1, 1)
        issue_rows(NCHUNK - 1, 1, 1)
        compute(0, 0)
        issue_scat(0, 0)

        wait_rows(1)
        wait_scat(0, 0)
        compute(1, 1)
        issue_scat(1, 1)
        wait_scat(1, 1)

        plsc.subcore_barrier()

        pltpu.sync_copy(out_sh.at[pl.ds(sid * RPT, RPT)],
                        outp_hbm.at[pl.ds(cid * N + sid * RPT, RPT)])

        @pl.when(sid == NS - 1)
        def _():
            pltpu.sync_copy(out_sh.at[pl.ds(NS * RPT, 16)],
                            outp_hbm.at[pl.ds(cid * N + NS * RPT, 16)])

        pltpu.sync_copy(den_v, denp_hbm.at[pl.ds(wid * N, N)])

    return body(xl, xr, e, sd, att)


def _comb_body(p_ref, den_ref, bias_ref, y_ref, ps_ref, pq_ref):
    p = p_ref[0] + p_ref[1]
    den = jnp.sum(den_ref[...], axis=1, keepdims=True)
    y = p / (den + 1e-16) + bias_ref[...]
    y_ref[...] = y
    ps_ref[...] = jnp.broadcast_to(jnp.sum(y, axis=0, keepdims=True), (8, C))
    pq_ref[...] = jnp.broadcast_to(jnp.sum(y * y, axis=0, keepdims=True), (8, C))


def _combine(p, denp, bias):
    blk = 1000
    nb = N // blk
    return pl.pallas_call(
        _comb_body,
        grid=(nb,),
        in_specs=[
            pl.BlockSpec((NC, blk, C), lambda i: (0, i, 0)),
            pl.BlockSpec((blk, NW), lambda i: (i, 0)),
            pl.BlockSpec((1, C), lambda i: (0, 0)),
        ],
        out_specs=[
            pl.BlockSpec((blk, C), lambda i: (i, 0)),
            pl.BlockSpec((8, C), lambda i: (i, 0)),
            pl.BlockSpec((8, C), lambda i: (i, 0)),
        ],
        out_shape=[
            jax.ShapeDtypeStruct((N, C), jnp.float32),
            jax.ShapeDtypeStruct((nb * 8, C), jnp.float32),
            jax.ShapeDtypeStruct((nb * 8, C), jnp.float32),
        ],
    )(p, denp, bias)


def _bn_body(y_ref, ps_ref, pq_ref, gamma_ref, beta_ref, out_ref):
    s = jnp.sum(ps_ref[...], axis=0, keepdims=True)
    sq = jnp.sum(pq_ref[...], axis=0, keepdims=True)
    mean = s / (8.0 * N)
    var = sq / (8.0 * N) - mean * mean
    inv = lax.rsqrt(var + 1e-5)
    o = (y_ref[...] - mean) * (inv * gamma_ref[...]) + beta_ref[...]
    out_ref[...] = jnp.where(o > 0, o, 0.01 * o)


def _batchnorm(y, ps, pq, gamma, beta):
    blk = 1000
    nb = N // blk
    return pl.pallas_call(
        _bn_body,
        grid=(nb,),
        in_specs=[
            pl.BlockSpec((blk, C), lambda i: (i, 0)),
            pl.BlockSpec((nb * 8, C), lambda i: (0, 0)),
            pl.BlockSpec((nb * 8, C), lambda i: (0, 0)),
            pl.BlockSpec((1, C), lambda i: (0, 0)),
            pl.BlockSpec((1, C), lambda i: (0, 0)),
        ],
        out_specs=pl.BlockSpec((blk, C), lambda i: (i, 0)),
        out_shape=jax.ShapeDtypeStruct((N, C), jnp.float32),
    )(y, ps, pq, gamma, beta)


def kernel(x, edge_index, edge_attr, W_l, b_l, W_r, b_r, W_e, att, bias, gamma, beta):
    xl, xr = _node_linears(x, W_l, W_r,
                           b_l.reshape(1, C), b_r.reshape(1, C))
    e = _edge_linear(edge_attr, W_e)
    sd = jnp.stack([edge_index[0].reshape(NW, NCHUNK, B),
                    edge_index[1].reshape(NW, NCHUNK, B)], axis=2)
    outp, denp = _sc_edge_pass(xl, xr, e, sd, att)
    y, ps, pq = _combine(outp.reshape(NC, N, C), denp.reshape(NW, N).T,
                         bias.reshape(1, C))
    return _batchnorm(y, ps, pq, gamma.reshape(1, C), beta.reshape(1, C))


# speedup vs baseline: 2.3182x; 1.1516x over previous
import functools

import jax
import jax.numpy as jnp
from jax import lax
from jax.experimental import pallas as pl
from jax.experimental.pallas import tpu as pltpu
from jax.experimental.pallas import tpu_sc as plsc

N = 10000
E = 320000
D = 128
C = 128
ED = 16

NC = 2
NS = 16
NW = NC * NS
EPT = E // NW
B = 40
NCHUNK = EPT // B
RPT = 624
ZR = 16
LANES = 16
G_STARTS = (0, 16, 24)


def _lin_body(x_ref, wl_ref, wr_ref, bl_ref, br_ref, xl_ref, xr_ref):
    x = x_ref[...]
    xl_ref[...] = jnp.dot(x, wl_ref[...], preferred_element_type=jnp.float32) + bl_ref[...]
    xr_ref[...] = jnp.dot(x, wr_ref[...], preferred_element_type=jnp.float32) + br_ref[...]


def _node_linears(x, W_l, W_r, b_l, b_r):
    blk = 1000
    return pl.pallas_call(
        _lin_body,
        grid=(N // blk,),
        in_specs=[
            pl.BlockSpec((blk, D), lambda i: (i, 0)),
            pl.BlockSpec((D, C), lambda i: (0, 0)),
            pl.BlockSpec((D, C), lambda i: (0, 0)),
            pl.BlockSpec((1, C), lambda i: (0, 0)),
            pl.BlockSpec((1, C), lambda i: (0, 0)),
        ],
        out_specs=[
            pl.BlockSpec((blk, C), lambda i: (i, 0)),
            pl.BlockSpec((blk, C), lambda i: (i, 0)),
        ],
        out_shape=[
            jax.ShapeDtypeStruct((N, C), jnp.float32),
            jax.ShapeDtypeStruct((N, C), jnp.float32),
        ],
    )(x, W_l, W_r, b_l, b_r)


def _edge_body(ea_ref, we_ref, e_ref):
    e_ref[...] = jnp.dot(ea_ref[...], we_ref[...], preferred_element_type=jnp.float32)


def _edge_linear(edge_attr, W_e):
    blk = 2000
    return pl.pallas_call(
        _edge_body,
        grid=(E // blk,),
        in_specs=[
            pl.BlockSpec((blk, ED), lambda i: (i, 0)),
            pl.BlockSpec((ED, C), lambda i: (0, 0)),
        ],
        out_specs=pl.BlockSpec((blk, C), lambda i: (i, 0)),
        out_shape=jax.ShapeDtypeStruct((E, C), jnp.float32),
    )(edge_attr, W_e)


def _sc_edge_pass(xl, xr, e, sd, att):
    mesh = plsc.VectorSubcoreMesh(core_axis_name="c", subcore_axis_name="s")

    @functools.partial(
        pl.kernel,
        mesh=mesh,
        compiler_params=pltpu.CompilerParams(needs_layout_passes=False),
        out_type=[
            jax.ShapeDtypeStruct((NC * N, C), jnp.float32),
            jax.ShapeDtypeStruct((NW * N,), jnp.float32),
        ],
        scratch_types=[
            pltpu.VMEM((4, 2, B), jnp.int32),
            pltpu.VMEM((B, C), jnp.float32),
            pltpu.VMEM((B, C), jnp.float32),
            pltpu.VMEM((B, C), jnp.float32),
            pltpu.VMEM((B, C), jnp.float32),
            pltpu.VMEM((B, C), jnp.float32),
            pltpu.VMEM((B, C), jnp.float32),
            pltpu.VMEM((B,), jnp.float32),
            pltpu.VMEM((B * LANES,), jnp.float32),
            pltpu.VMEM((C,), jnp.float32),
            pltpu.VMEM((N,), jnp.float32),
            pltpu.VMEM((ZR, C), jnp.float32),
            pltpu.VMEM_SHARED((N, C), jnp.float32),
            pltpu.SemaphoreType.DMA,
            pltpu.SemaphoreType.DMA,
            pltpu.SemaphoreType.DMA,
            pltpu.SemaphoreType.DMA,
            pltpu.SemaphoreType.DMA,
        ],
    )
    def body(xl_hbm, xr_hbm, e_hbm, sd_hbm, att_hbm,
             outp_hbm, denp_hbm,
             idx2, rl0, rl1, rr0, rr1, re0, re1, ea_v, accb_v, att_v,
             den_v, z_v, out_sh, gsem0, gsem1, ssem, isem0, isem1):
        cid = lax.axis_index("c")
        sid = lax.axis_index("s")
        wid = cid * NS + sid
        rl = (rl0, rl1)
        rr = (rr0, rr1)
        re = (re0, re1)
        gsem = (gsem0, gsem1)

        pltpu.sync_copy(att_hbm, att_v)

        def zero_z(r, _):
            for j in range(C // LANES):
                z_v[r, pl.ds(j * LANES, LANES)] = jnp.zeros((LANES,), jnp.float32)
            return _
        lax.fori_loop(0, ZR, zero_z, None)

        def zero_den(g, _):
            den_v[pl.ds(g * LANES, LANES)] = jnp.zeros((LANES,), jnp.float32)
            return _
        lax.fori_loop(0, N // LANES, zero_den, None)

        def zero_sh(k, _):
            pltpu.sync_copy(z_v, out_sh.at[pl.ds(sid * RPT + k * ZR, ZR)])
            return _
        lax.fori_loop(0, RPT // ZR, zero_sh, None)

        @pl.when(sid == NS - 1)
        def _():
            pltpu.sync_copy(z_v.at[pl.ds(0, 16)],
                            out_sh.at[pl.ds(NS * RPT, 16)])

        plsc.subcore_barrier()

        isem = (isem0, isem1)

        def fetch_idx(c, s, par):
            pltpu.async_copy(sd_hbm.at[wid, c], idx2.at[s], isem[par])

        def wait_idx(s, par):
            pltpu.make_async_copy(sd_hbm.at[wid, 0], idx2.at[s],
                                  isem[par]).wait()

        def issue_rows(c, s, p):
            pltpu.async_copy(xl_hbm.at[idx2.at[s, 0]], rl[p], gsem[p])
            pltpu.async_copy(xr_hbm.at[idx2.at[s, 1]], rr[p], gsem[p])
            pltpu.async_copy(e_hbm.at[pl.ds(wid * EPT + c * B, B)],
                             re[p], gsem[p])

        def wait_rows(p):
            pltpu.make_async_copy(e_hbm.at[pl.ds(0, B)], rl[p], gsem[p]).wait()
            pltpu.make_async_copy(e_hbm.at[pl.ds(0, B)], rr[p], gsem[p]).wait()
            pltpu.make_async_copy(e_hbm.at[pl.ds(0, B)], re[p], gsem[p]).wait()

        def issue_scat(s, p):
            pltpu.async_copy(rl[p], out_sh.at[idx2.at[s, 1]], ssem,
                             add=True)

        def wait_scat(s, p):
            pltpu.make_async_copy(rl[p], out_sh.at[idx2.at[s, 1]],
                                  ssem).wait()

        lane = lax.iota(jnp.int32, LANES)
        tail_mask = lane >= (G_STARTS[-2] + LANES - G_STARTS[-1])

        def compute(s, p):
            rl_v, rr_v, re_v = rl[p], rr[p], re[p]

            def edge_alpha(eg, _):
                acc = jnp.zeros((LANES,), jnp.float32)
                for j in range(C // LANES):
                    sl = pl.ds(j * LANES, LANES)
                    m = rl_v[eg, sl] + rr_v[eg, sl] + re_v[eg, sl]
                    m = jnp.maximum(m, 0.2 * m)
                    acc = acc + m * att_v[sl]
                accb_v[pl.ds(eg * LANES, LANES)] = acc
                return _
            lax.fori_loop(0, B, edge_alpha, None)

            for g0 in G_STARTS:
                sl = pl.ds(g0, LANES)
                flat = (lane + g0) * LANES
                asum = jnp.zeros((LANES,), jnp.float32)
                for cc in range(LANES):
                    asum = asum + plsc.load_gather(accb_v, [flat + cc])
                eav = jnp.exp(asum)
                ea_v[sl] = eav
                mask = tail_mask if g0 == G_STARTS[-1] else None
                plsc.addupdate_scatter(den_v, [idx2[s, 1, sl]], eav,
                                       mask=mask)

            def edge_scale(eg, _):
                s = plsc.load_gather(ea_v, [jnp.full((LANES,), eg, jnp.int32)])
                for j in range(C // LANES):
                    sl = pl.ds(j * LANES, LANES)
                    rl_v[eg, sl] = rl_v[eg, sl] * s
                return _
            lax.fori_loop(0, B, edge_scale, None)

        fetch_idx(0, 0, 0)
        wait_idx(0, 0)
        issue_rows(0, 0, 0)
        fetch_idx(1, 1, 1)

        def pipe_body(k):
            for b in range(4):
                c = k + b
                s = b
                sn = (b + 1) % 4
                sf = (b + 2) % 4
                p = b % 2
                q = 1 - p
                wait_rows(p)

                @pl.when(c >= 1)
                def _():
                    wait_scat((b + 3) % 4, q)
                wait_idx(sn, (b + 1) % 2)
                issue_rows(c + 1, sn, q)
                fetch_idx(c + 2, sf, b % 2)
                compute(s, p)
                issue_scat(s, p)

        pl.loop(0, NCHUNK - 2, step=4)(pipe_body)

        wait_rows(0)
        wait_scat(3, 1)
        wait_idx(1, 1)
        issue_rows(NCHUNK - 1, 1, 1)
        compute(0, 0)
        issue_scat(0, 0)

        wait_rows(1)
        wait_scat(0, 0)
        compute(1, 1)
        issue_scat(1, 1)
        wait_scat(1, 1)

        plsc.subcore_barrier()

        pltpu.sync_copy(out_sh.at[pl.ds(sid * RPT, RPT)],
                        outp_hbm.at[pl.ds(cid * N + sid * RPT, RPT)])

        @pl.when(sid == NS - 1)
        def _():
            pltpu.sync_copy(out_sh.at[pl.ds(NS * RPT, 16)],
                            outp_hbm.at[pl.ds(cid * N + NS * RPT, 16)])

        pltpu.sync_copy(den_v, denp_hbm.at[pl.ds(wid * N, N)])

    return body(xl, xr, e, sd, att)


_CB_BLK = 1000
_CB_NB = N // _CB_BLK


def _comb_bn_body(p_ref, den_ref, bias_ref, gamma_ref, beta_ref, out_ref,
                  y_s, ps_s, pq_s):
    i = pl.program_id(0)

    @pl.when(i < _CB_NB)
    def _():
        p = p_ref[0] + p_ref[1]
        den = jnp.sum(den_ref[...], axis=1, keepdims=True)
        y = p / (den + 1e-16) + bias_ref[...]
        y_s[pl.ds(i * _CB_BLK, _CB_BLK), :] = y
        ps = jnp.sum(y, axis=0, keepdims=True)
        pq = jnp.sum(y * y, axis=0, keepdims=True)

        @pl.when(i == 0)
        def _():
            ps_s[...] = ps
            pq_s[...] = pq

        @pl.when(i > 0)
        def _():
            ps_s[...] = ps_s[...] + ps
            pq_s[...] = pq_s[...] + pq

    @pl.when(i >= _CB_NB)
    def _():
        j = i - _CB_NB
        mean = ps_s[...] / N
        var = pq_s[...] / N - mean * mean
        inv = lax.rsqrt(var + 1e-5)
        y = y_s[pl.ds(j * _CB_BLK, _CB_BLK), :]
        o = (y - mean) * (inv * gamma_ref[...]) + beta_ref[...]
        out_ref[...] = jnp.where(o > 0, o, 0.01 * o)


def _combine_bn(p, denp, bias, gamma, beta):
    nb = _CB_NB
    blk = _CB_BLK
    return pl.pallas_call(
        _comb_bn_body,
        grid=(2 * nb,),
        in_specs=[
            pl.BlockSpec((NC, blk, C),
                         lambda i: (0, jnp.minimum(i, nb - 1), 0)),
            pl.BlockSpec((blk, NW), lambda i: (jnp.minimum(i, nb - 1), 0)),
            pl.BlockSpec((1, C), lambda i: (0, 0)),
            pl.BlockSpec((1, C), lambda i: (0, 0)),
            pl.BlockSpec((1, C), lambda i: (0, 0)),
        ],
        out_specs=pl.BlockSpec((blk, C), lambda i: (jnp.maximum(i - nb, 0), 0)),
        out_shape=jax.ShapeDtypeStruct((N, C), jnp.float32),
        scratch_shapes=[
            pltpu.VMEM((N, C), jnp.float32),
            pltpu.VMEM((1, C), jnp.float32),
            pltpu.VMEM((1, C), jnp.float32),
        ],
    )(p, denp, bias, gamma, beta)


def kernel(x, edge_index, edge_attr, W_l, b_l, W_r, b_r, W_e, att, bias, gamma, beta):
    xl, xr = _node_linears(x, W_l, W_r,
                           b_l.reshape(1, C), b_r.reshape(1, C))
    e = _edge_linear(edge_attr, W_e)
    sd = jnp.stack([edge_index[0].reshape(NW, NCHUNK, B),
                    edge_index[1].reshape(NW, NCHUNK, B)], axis=2)
    outp, denp = _sc_edge_pass(xl, xr, e, sd, att)
    return _combine_bn(outp.reshape(NC, N, C), denp.reshape(NW, N).T,
                       bias.reshape(1, C), gamma.reshape(1, C),
                       beta.reshape(1, C))


# speedup vs baseline: 2.5179x; 1.0861x over previous
import functools

import jax
import jax.numpy as jnp
from jax import lax
from jax.experimental import pallas as pl
from jax.experimental.pallas import tpu as pltpu
from jax.experimental.pallas import tpu_sc as plsc

N = 10000
E = 320000
D = 128
C = 128
ED = 16

NC = 2
NS = 16
NW = NC * NS
EPT = E // NW
B = 40
NCHUNK = EPT // B
RPT = 624
ZR = 16
LANES = 16
G_STARTS = (0, 16, 24)


def _lin_body(x_ref, wl_ref, wr_ref, bl_ref, br_ref, xl_ref, xr_ref):
    x = x_ref[...]
    xl_ref[...] = jnp.dot(x, wl_ref[...], preferred_element_type=jnp.float32) + bl_ref[...]
    xr_ref[...] = jnp.dot(x, wr_ref[...], preferred_element_type=jnp.float32) + br_ref[...]


def _node_linears(x, W_l, W_r, b_l, b_r):
    blk = 1000
    return pl.pallas_call(
        _lin_body,
        grid=(N // blk,),
        in_specs=[
            pl.BlockSpec((blk, D), lambda i: (i, 0)),
            pl.BlockSpec((D, C), lambda i: (0, 0)),
            pl.BlockSpec((D, C), lambda i: (0, 0)),
            pl.BlockSpec((1, C), lambda i: (0, 0)),
            pl.BlockSpec((1, C), lambda i: (0, 0)),
        ],
        out_specs=[
            pl.BlockSpec((blk, C), lambda i: (i, 0)),
            pl.BlockSpec((blk, C), lambda i: (i, 0)),
        ],
        out_shape=[
            jax.ShapeDtypeStruct((N, C), jnp.float32),
            jax.ShapeDtypeStruct((N, C), jnp.float32),
        ],
    )(x, W_l, W_r, b_l, b_r)


def _edge_body(ea_ref, we_ref, e_ref):
    e_ref[...] = jnp.dot(ea_ref[...], we_ref[...], preferred_element_type=jnp.float32)


def _edge_linear(edge_attr, W_e):
    blk = 8000
    return pl.pallas_call(
        _edge_body,
        grid=(E // blk,),
        in_specs=[
            pl.BlockSpec((blk, ED), lambda i: (i, 0)),
            pl.BlockSpec((ED, C), lambda i: (0, 0)),
        ],
        out_specs=pl.BlockSpec((blk, C), lambda i: (i, 0)),
        out_shape=jax.ShapeDtypeStruct((E, C), jnp.float32),
    )(edge_attr, W_e)


def _sc_edge_pass(xl, xr, e, sd, att):
    mesh = plsc.VectorSubcoreMesh(core_axis_name="c", subcore_axis_name="s")

    @functools.partial(
        pl.kernel,
        mesh=mesh,
        compiler_params=pltpu.CompilerParams(needs_layout_passes=False),
        out_type=[
            jax.ShapeDtypeStruct((NC * N, C), jnp.float32),
            jax.ShapeDtypeStruct((NW * N,), jnp.float32),
        ],
        scratch_types=[
            pltpu.VMEM((4, 2, B), jnp.int32),
            pltpu.VMEM((B, C), jnp.float32),
            pltpu.VMEM((B, C), jnp.float32),
            pltpu.VMEM((B, C), jnp.float32),
            pltpu.VMEM((B, C), jnp.float32),
            pltpu.VMEM((B, C), jnp.float32),
            pltpu.VMEM((B, C), jnp.float32),
            pltpu.VMEM((B,), jnp.float32),
            pltpu.VMEM((B * LANES,), jnp.float32),
            pltpu.VMEM((C,), jnp.float32),
            pltpu.VMEM((N,), jnp.float32),
            pltpu.VMEM((ZR, C), jnp.float32),
            pltpu.VMEM_SHARED((N, C), jnp.float32),
            pltpu.SemaphoreType.DMA,
            pltpu.SemaphoreType.DMA,
            pltpu.SemaphoreType.DMA,
            pltpu.SemaphoreType.DMA,
            pltpu.SemaphoreType.DMA,
        ],
    )
    def body(xl_hbm, xr_hbm, e_hbm, sd_hbm, att_hbm,
             outp_hbm, denp_hbm,
             idx2, rl0, rl1, rr0, rr1, re0, re1, ea_v, accb_v, att_v,
             den_v, z_v, out_sh, gsem0, gsem1, ssem, isem0, isem1):
        cid = lax.axis_index("c")
        sid = lax.axis_index("s")
        wid = cid * NS + sid
        rl = (rl0, rl1)
        rr = (rr0, rr1)
        re = (re0, re1)
        gsem = (gsem0, gsem1)

        pltpu.sync_copy(att_hbm, att_v)

        def zero_z(r, _):
            for j in range(C // LANES):
                z_v[r, pl.ds(j * LANES, LANES)] = jnp.zeros((LANES,), jnp.float32)
            return _
        lax.fori_loop(0, ZR, zero_z, None)

        def zero_den(g, _):
            den_v[pl.ds(g * LANES, LANES)] = jnp.zeros((LANES,), jnp.float32)
            return _
        lax.fori_loop(0, N // LANES, zero_den, None)

        def zero_sh(k, _):
            pltpu.sync_copy(z_v, out_sh.at[pl.ds(sid * RPT + k * ZR, ZR)])
            return _
        lax.fori_loop(0, RPT // ZR, zero_sh, None)

        @pl.when(sid == NS - 1)
        def _():
            pltpu.sync_copy(z_v.at[pl.ds(0, 16)],
                            out_sh.at[pl.ds(NS * RPT, 16)])

        plsc.subcore_barrier()

        isem = (isem0, isem1)

        def fetch_idx(c, s, par):
            pltpu.async_copy(sd_hbm.at[wid, c], idx2.at[s], isem[par])

        def wait_idx(s, par):
            pltpu.make_async_copy(sd_hbm.at[wid, 0], idx2.at[s],
                                  isem[par]).wait()

        def issue_rows(c, s, p):
            pltpu.async_copy(xl_hbm.at[idx2.at[s, 0]], rl[p], gsem[p])
            pltpu.async_copy(xr_hbm.at[idx2.at[s, 1]], rr[p], gsem[p])
            pltpu.async_copy(e_hbm.at[pl.ds(wid * EPT + c * B, B)],
                             re[p], gsem[p])

        def wait_rows(p):
            pltpu.make_async_copy(e_hbm.at[pl.ds(0, B)], rl[p], gsem[p]).wait()
            pltpu.make_async_copy(e_hbm.at[pl.ds(0, B)], rr[p], gsem[p]).wait()
            pltpu.make_async_copy(e_hbm.at[pl.ds(0, B)], re[p], gsem[p]).wait()

        def issue_scat(s, p):
            pltpu.async_copy(rl[p], out_sh.at[idx2.at[s, 1]], ssem,
                             add=True)

        def wait_scat(s, p):
            pltpu.make_async_copy(rl[p], out_sh.at[idx2.at[s, 1]],
                                  ssem).wait()

        lane = lax.iota(jnp.int32, LANES)
        tail_mask = lane >= (G_STARTS[-2] + LANES - G_STARTS[-1])

        def compute(s, p):
            rl_v, rr_v, re_v = rl[p], rr[p], re[p]

            def edge_alpha(eg, _):
                acc = jnp.zeros((LANES,), jnp.float32)
                for j in range(C // LANES):
                    sl = pl.ds(j * LANES, LANES)
                    m = rl_v[eg, sl] + rr_v[eg, sl] + re_v[eg, sl]
                    m = jnp.maximum(m, 0.2 * m)
                    acc = acc + m * att_v[sl]
                accb_v[pl.ds(eg * LANES, LANES)] = acc
                return _
            lax.fori_loop(0, B, edge_alpha, None)

            for g0 in G_STARTS:
                sl = pl.ds(g0, LANES)
                flat = (lane + g0) * LANES
                asum = jnp.zeros((LANES,), jnp.float32)
                for cc in range(LANES):
                    asum = asum + plsc.load_gather(accb_v, [flat + cc])
                eav = jnp.exp(asum)
                ea_v[sl] = eav
                mask = tail_mask if g0 == G_STARTS[-1] else None
                plsc.addupdate_scatter(den_v, [idx2[s, 1, sl]], eav,
                                       mask=mask)

            def edge_scale(eg, _):
                s = plsc.load_gather(ea_v, [jnp.full((LANES,), eg, jnp.int32)])
                for j in range(C // LANES):
                    sl = pl.ds(j * LANES, LANES)
                    rl_v[eg, sl] = rl_v[eg, sl] * s
                return _
            lax.fori_loop(0, B, edge_scale, None)

        fetch_idx(0, 0, 0)
        wait_idx(0, 0)
        issue_rows(0, 0, 0)
        fetch_idx(1, 1, 1)

        def pipe_body(k):
            for b in range(4):
                c = k + b
                s = b
                sn = (b + 1) % 4
                sf = (b + 2) % 4
                p = b % 2
                q = 1 - p
                wait_rows(p)

                @pl.when(c >= 1)
                def _():
                    wait_scat((b + 3) % 4, q)
                wait_idx(sn, (b + 1) % 2)
                issue_rows(c + 1, sn, q)
                fetch_idx(c + 2, sf, b % 2)
                compute(s, p)
                issue_scat(s, p)

        pl.loop(0, NCHUNK - 2, step=4)(pipe_body)

        wait_rows(0)
        wait_scat(3, 1)
        wait_idx(1, 1)
        issue_rows(NCHUNK - 1, 1, 1)
        compute(0, 0)
        issue_scat(0, 0)

        wait_rows(1)
        wait_scat(0, 0)
        compute(1, 1)
        issue_scat(1, 1)
        wait_scat(1, 1)

        plsc.subcore_barrier()

        pltpu.sync_copy(out_sh.at[pl.ds(sid * RPT, RPT)],
                        outp_hbm.at[pl.ds(cid * N + sid * RPT, RPT)])

        @pl.when(sid == NS - 1)
        def _():
            pltpu.sync_copy(out_sh.at[pl.ds(NS * RPT, 16)],
                            outp_hbm.at[pl.ds(cid * N + NS * RPT, 16)])

        pltpu.sync_copy(den_v, denp_hbm.at[pl.ds(wid * N, N)])

    return body(xl, xr, e, sd, att)


_CB_BLK = 1000
_CB_NB = N // _CB_BLK


def _comb_bn_body(p_ref, den_ref, bias_ref, gamma_ref, beta_ref, out_ref,
                  y_s, ps_s, pq_s):
    i = pl.program_id(0)

    @pl.when(i < _CB_NB)
    def _():
        p = p_ref[0] + p_ref[1]
        den = jnp.sum(den_ref[...], axis=1, keepdims=True)
        y = p / (den + 1e-16) + bias_ref[...]
        y_s[pl.ds(i * _CB_BLK, _CB_BLK), :] = y
        ps = jnp.sum(y, axis=0, keepdims=True)
        pq = jnp.sum(y * y, axis=0, keepdims=True)

        @pl.when(i == 0)
        def _():
            ps_s[...] = ps
            pq_s[...] = pq

        @pl.when(i > 0)
        def _():
            ps_s[...] = ps_s[...] + ps
            pq_s[...] = pq_s[...] + pq

    @pl.when(i >= _CB_NB)
    def _():
        j = i - _CB_NB
        mean = ps_s[...] / N
        var = pq_s[...] / N - mean * mean
        inv = lax.rsqrt(var + 1e-5)
        y = y_s[pl.ds(j * _CB_BLK, _CB_BLK), :]
        o = (y - mean) * (inv * gamma_ref[...]) + beta_ref[...]
        out_ref[...] = jnp.where(o > 0, o, 0.01 * o)


def _combine_bn(p, denp, bias, gamma, beta):
    nb = _CB_NB
    blk = _CB_BLK
    return pl.pallas_call(
        _comb_bn_body,
        grid=(2 * nb,),
        in_specs=[
            pl.BlockSpec((NC, blk, C),
                         lambda i: (0, jnp.minimum(i, nb - 1), 0)),
            pl.BlockSpec((blk, NW), lambda i: (jnp.minimum(i, nb - 1), 0)),
            pl.BlockSpec((1, C), lambda i: (0, 0)),
            pl.BlockSpec((1, C), lambda i: (0, 0)),
            pl.BlockSpec((1, C), lambda i: (0, 0)),
        ],
        out_specs=pl.BlockSpec((blk, C), lambda i: (jnp.maximum(i - nb, 0), 0)),
        out_shape=jax.ShapeDtypeStruct((N, C), jnp.float32),
        scratch_shapes=[
            pltpu.VMEM((N, C), jnp.float32),
            pltpu.VMEM((1, C), jnp.float32),
            pltpu.VMEM((1, C), jnp.float32),
        ],
    )(p, denp, bias, gamma, beta)


def kernel(x, edge_index, edge_attr, W_l, b_l, W_r, b_r, W_e, att, bias, gamma, beta):
    xl, xr = _node_linears(x, W_l, W_r,
                           b_l.reshape(1, C), b_r.reshape(1, C))
    e = _edge_linear(edge_attr, W_e)
    sd = jnp.stack([edge_index[0].reshape(NW, NCHUNK, B),
                    edge_index[1].reshape(NW, NCHUNK, B)], axis=2)
    outp, denp = _sc_edge_pass(xl, xr, e, sd, att)
    return _combine_bn(outp.reshape(NC, N, C), denp.reshape(NW, N).T,
                       bias.reshape(1, C), gamma.reshape(1, C),
                       beta.reshape(1, C))


# speedup vs baseline: 2.5347x; 1.0067x over previous
import functools

import jax
import jax.numpy as jnp
from jax import lax
from jax.experimental import pallas as pl
from jax.experimental.pallas import tpu as pltpu
from jax.experimental.pallas import tpu_sc as plsc

N = 10000
E = 320000
D = 128
C = 128
ED = 16

NC = 2
NS = 16
NW = NC * NS
EPT = E // NW
B = 40
NCHUNK = EPT // B
RPT = 624
ZR = 16
LANES = 16
G_STARTS = (0, 16, 24)


def _lin_body(x_ref, wl_ref, wr_ref, bl_ref, br_ref, xl_ref, xr_ref):
    x = x_ref[...]
    xl_ref[...] = jnp.dot(x, wl_ref[...], preferred_element_type=jnp.float32) + bl_ref[...]
    xr_ref[...] = jnp.dot(x, wr_ref[...], preferred_element_type=jnp.float32) + br_ref[...]


def _node_linears(x, W_l, W_r, b_l, b_r):
    blk = 2000
    return pl.pallas_call(
        _lin_body,
        grid=(N // blk,),
        in_specs=[
            pl.BlockSpec((blk, D), lambda i: (i, 0)),
            pl.BlockSpec((D, C), lambda i: (0, 0)),
            pl.BlockSpec((D, C), lambda i: (0, 0)),
            pl.BlockSpec((1, C), lambda i: (0, 0)),
            pl.BlockSpec((1, C), lambda i: (0, 0)),
        ],
        out_specs=[
            pl.BlockSpec((blk, C), lambda i: (i, 0)),
            pl.BlockSpec((blk, C), lambda i: (i, 0)),
        ],
        out_shape=[
            jax.ShapeDtypeStruct((N, C), jnp.float32),
            jax.ShapeDtypeStruct((N, C), jnp.float32),
        ],
    )(x, W_l, W_r, b_l, b_r)


def _edge_body(ea_ref, we_ref, e_ref):
    e_ref[...] = jnp.dot(ea_ref[...], we_ref[...], preferred_element_type=jnp.float32)


def _edge_linear(edge_attr, W_e):
    blk = 16000
    return pl.pallas_call(
        _edge_body,
        grid=(E // blk,),
        in_specs=[
            pl.BlockSpec((blk, ED), lambda i: (i, 0)),
            pl.BlockSpec((ED, C), lambda i: (0, 0)),
        ],
        out_specs=pl.BlockSpec((blk, C), lambda i: (i, 0)),
        out_shape=jax.ShapeDtypeStruct((E, C), jnp.float32),
    )(edge_attr, W_e)


def _sc_edge_pass(xl, xr, e, sd, att):
    mesh = plsc.VectorSubcoreMesh(core_axis_name="c", subcore_axis_name="s")

    @functools.partial(
        pl.kernel,
        mesh=mesh,
        compiler_params=pltpu.CompilerParams(needs_layout_passes=False),
        out_type=[
            jax.ShapeDtypeStruct((NC * N, C), jnp.float32),
            jax.ShapeDtypeStruct((NW * N,), jnp.float32),
        ],
        scratch_types=[
            pltpu.VMEM((4, 2, B), jnp.int32),
            pltpu.VMEM((B, C), jnp.float32),
            pltpu.VMEM((B, C), jnp.float32),
            pltpu.VMEM((B, C), jnp.float32),
            pltpu.VMEM((B, C), jnp.float32),
            pltpu.VMEM((B, C), jnp.float32),
            pltpu.VMEM((B, C), jnp.float32),
            pltpu.VMEM((B,), jnp.float32),
            pltpu.VMEM((B * LANES,), jnp.float32),
            pltpu.VMEM((C,), jnp.float32),
            pltpu.VMEM((N,), jnp.float32),
            pltpu.VMEM((ZR, C), jnp.float32),
            pltpu.VMEM_SHARED((N, C), jnp.float32),
            pltpu.SemaphoreType.DMA,
            pltpu.SemaphoreType.DMA,
            pltpu.SemaphoreType.DMA,
            pltpu.SemaphoreType.DMA,
            pltpu.SemaphoreType.DMA,
        ],
    )
    def body(xl_hbm, xr_hbm, e_hbm, sd_hbm, att_hbm,
             outp_hbm, denp_hbm,
             idx2, rl0, rl1, rr0, rr1, re0, re1, ea_v, accb_v, att_v,
             den_v, z_v, out_sh, gsem0, gsem1, ssem, isem0, isem1):
        cid = lax.axis_index("c")
        sid = lax.axis_index("s")
        wid = cid * NS + sid
        rl = (rl0, rl1)
        rr = (rr0, rr1)
        re = (re0, re1)
        gsem = (gsem0, gsem1)

        pltpu.sync_copy(att_hbm, att_v)

        def zero_z(r, _):
            for j in range(C // LANES):
                z_v[r, pl.ds(j * LANES, LANES)] = jnp.zeros((LANES,), jnp.float32)
            return _
        lax.fori_loop(0, ZR, zero_z, None)

        def zero_den(g, _):
            den_v[pl.ds(g * LANES, LANES)] = jnp.zeros((LANES,), jnp.float32)
            return _
        lax.fori_loop(0, N // LANES, zero_den, None)

        def zero_sh(k, _):
            pltpu.sync_copy(z_v, out_sh.at[pl.ds(sid * RPT + k * ZR, ZR)])
            return _
        lax.fori_loop(0, RPT // ZR, zero_sh, None)

        @pl.when(sid == NS - 1)
        def _():
            pltpu.sync_copy(z_v.at[pl.ds(0, 16)],
                            out_sh.at[pl.ds(NS * RPT, 16)])

        plsc.subcore_barrier()

        isem = (isem0, isem1)

        def fetch_idx(c, s, par):
            pltpu.async_copy(sd_hbm.at[wid, c], idx2.at[s], isem[par])

        def wait_idx(s, par):
            pltpu.make_async_copy(sd_hbm.at[wid, 0], idx2.at[s],
                                  isem[par]).wait()

        def issue_rows(c, s, p):
            pltpu.async_copy(xl_hbm.at[idx2.at[s, 0]], rl[p], gsem[p])
            pltpu.async_copy(xr_hbm.at[idx2.at[s, 1]], rr[p], gsem[p])
            pltpu.async_copy(e_hbm.at[pl.ds(wid * EPT + c * B, B)],
                             re[p], gsem[p])

        def wait_rows(p):
            pltpu.make_async_copy(e_hbm.at[pl.ds(0, B)], rl[p], gsem[p]).wait()
            pltpu.make_async_copy(e_hbm.at[pl.ds(0, B)], rr[p], gsem[p]).wait()
            pltpu.make_async_copy(e_hbm.at[pl.ds(0, B)], re[p], gsem[p]).wait()

        def issue_scat(s, p):
            pltpu.async_copy(rl[p], out_sh.at[idx2.at[s, 1]], ssem,
                             add=True)

        def wait_scat(s, p):
            pltpu.make_async_copy(rl[p], out_sh.at[idx2.at[s, 1]],
                                  ssem).wait()

        lane = lax.iota(jnp.int32, LANES)
        tail_mask = lane >= (G_STARTS[-2] + LANES - G_STARTS[-1])

        def compute(s, p):
            rl_v, rr_v, re_v = rl[p], rr[p], re[p]

            def edge_alpha(eg, _):
                acc = jnp.zeros((LANES,), jnp.float32)
                for j in range(C // LANES):
                    sl = pl.ds(j * LANES, LANES)
                    m = rl_v[eg, sl] + rr_v[eg, sl] + re_v[eg, sl]
                    m = jnp.maximum(m, 0.2 * m)
                    acc = acc + m * att_v[sl]
                accb_v[pl.ds(eg * LANES, LANES)] = acc
                return _
            lax.fori_loop(0, B, edge_alpha, None)

            for g0 in G_STARTS:
                sl = pl.ds(g0, LANES)
                flat = (lane + g0) * LANES
                asum = jnp.zeros((LANES,), jnp.float32)
                for cc in range(LANES):
                    asum = asum + plsc.load_gather(accb_v, [flat + cc])
                eav = jnp.exp(asum)
                ea_v[sl] = eav
                mask = tail_mask if g0 == G_STARTS[-1] else None
                plsc.addupdate_scatter(den_v, [idx2[s, 1, sl]], eav,
                                       mask=mask)

            def edge_scale(eg, _):
                s = plsc.load_gather(ea_v, [jnp.full((LANES,), eg, jnp.int32)])
                for j in range(C // LANES):
                    sl = pl.ds(j * LANES, LANES)
                    rl_v[eg, sl] = rl_v[eg, sl] * s
                return _
            lax.fori_loop(0, B, edge_scale, None)

        fetch_idx(0, 0, 0)
        wait_idx(0, 0)
        issue_rows(0, 0, 0)
        fetch_idx(1, 1, 1)

        def pipe_body(k):
            for b in range(4):
                c = k + b
                s = b
                sn = (b + 1) % 4
                sf = (b + 2) % 4
                p = b % 2
                q = 1 - p
                wait_rows(p)

                @pl.when(c >= 1)
                def _():
                    wait_scat((b + 3) % 4, q)
                wait_idx(sn, (b + 1) % 2)
                issue_rows(c + 1, sn, q)
                fetch_idx(c + 2, sf, b % 2)
                compute(s, p)
                issue_scat(s, p)

        pl.loop(0, NCHUNK - 2, step=4)(pipe_body)

        wait_rows(0)
        wait_scat(3, 1)
        wait_idx(1, 1)
        issue_rows(NCHUNK - 1, 1, 1)
        compute(0, 0)
        issue_scat(0, 0)

        wait_rows(1)
        wait_scat(0, 0)
        compute(1, 1)
        issue_scat(1, 1)
        wait_scat(1, 1)

        plsc.subcore_barrier()

        pltpu.sync_copy(out_sh.at[pl.ds(sid * RPT, RPT)],
                        outp_hbm.at[pl.ds(cid * N + sid * RPT, RPT)])

        @pl.when(sid == NS - 1)
        def _():
            pltpu.sync_copy(out_sh.at[pl.ds(NS * RPT, 16)],
                            outp_hbm.at[pl.ds(cid * N + NS * RPT, 16)])

        pltpu.sync_copy(den_v, denp_hbm.at[pl.ds(wid * N, N)])

    return body(xl, xr, e, sd, att)


_CB_BLK = 1000
_CB_NB = N // _CB_BLK


def _comb_bn_body(p_ref, den_ref, bias_ref, gamma_ref, beta_ref, out_ref,
                  y_s, ps_s, pq_s):
    i = pl.program_id(0)

    @pl.when(i < _CB_NB)
    def _():
        p = p_ref[0] + p_ref[1]
        den = jnp.sum(den_ref[...], axis=1, keepdims=True)
        y = p / (den + 1e-16) + bias_ref[...]
        y_s[pl.ds(i * _CB_BLK, _CB_BLK), :] = y
        ps = jnp.sum(y, axis=0, keepdims=True)
        pq = jnp.sum(y * y, axis=0, keepdims=True)

        @pl.when(i == 0)
        def _():
            ps_s[...] = ps
            pq_s[...] = pq

        @pl.when(i > 0)
        def _():
            ps_s[...] = ps_s[...] + ps
            pq_s[...] = pq_s[...] + pq

    @pl.when(i >= _CB_NB)
    def _():
        j = i - _CB_NB
        mean = ps_s[...] / N
        var = pq_s[...] / N - mean * mean
        inv = lax.rsqrt(var + 1e-5)
        y = y_s[pl.ds(j * _CB_BLK, _CB_BLK), :]
        o = (y - mean) * (inv * gamma_ref[...]) + beta_ref[...]
        out_ref[...] = jnp.where(o > 0, o, 0.01 * o)


def _combine_bn(p, denp, bias, gamma, beta):
    nb = _CB_NB
    blk = _CB_BLK
    return pl.pallas_call(
        _comb_bn_body,
        grid=(2 * nb,),
        in_specs=[
            pl.BlockSpec((NC, blk, C),
                         lambda i: (0, jnp.minimum(i, nb - 1), 0)),
            pl.BlockSpec((blk, NW), lambda i: (jnp.minimum(i, nb - 1), 0)),
            pl.BlockSpec((1, C), lambda i: (0, 0)),
            pl.BlockSpec((1, C), lambda i: (0, 0)),
            pl.BlockSpec((1, C), lambda i: (0, 0)),
        ],
        out_specs=pl.BlockSpec((blk, C), lambda i: (jnp.maximum(i - nb, 0), 0)),
        out_shape=jax.ShapeDtypeStruct((N, C), jnp.float32),
        scratch_shapes=[
            pltpu.VMEM((N, C), jnp.float32),
            pltpu.VMEM((1, C), jnp.float32),
            pltpu.VMEM((1, C), jnp.float32),
        ],
    )(p, denp, bias, gamma, beta)


def kernel(x, edge_index, edge_attr, W_l, b_l, W_r, b_r, W_e, att, bias, gamma, beta):
    xl, xr = _node_linears(x, W_l, W_r,
                           b_l.reshape(1, C), b_r.reshape(1, C))
    e = _edge_linear(edge_attr, W_e)
    sd = jnp.stack([edge_index[0].reshape(NW, NCHUNK, B),
                    edge_index[1].reshape(NW, NCHUNK, B)], axis=2)
    outp, denp = _sc_edge_pass(xl, xr, e, sd, att)
    return _combine_bn(outp.reshape(NC, N, C), denp.reshape(NW, N).T,
                       bias.reshape(1, C), gamma.reshape(1, C),
                       beta.reshape(1, C))
